# Initial kernel scaffold; baseline (speedup 1.0000x reference)
import jax
import jax.numpy as jnp
from jax.experimental import pallas as pl


def kernel(x, conv1_w, bn1_g, bn1_b, conv2_w, bn2_g, bn2_b, c1_w1, c1_b1, c1_w2, c1_b2, c1_g, c1_be, c2_w1, c2_b1, c2_w2, c2_b2, c2_g, c2_be, c3_w1, c3_b1, c3_w2, c3_b2, c3_g, c3_be, lin1_w, lin1_b, lin2_w, lin2_b):
    raise NotImplementedError("write your pallas kernel here")


# speedup vs baseline: 3.1045x; 3.1045x over previous
import numpy as np
import jax
import jax.numpy as jnp
from jax.experimental import pallas as pl
from jax.experimental.pallas import tpu as pltpu

_B = 64
_C = 30
_T = 3600
_NSEG = 9
_SEG = _T // _NSEG
_K1 = 64
_L1 = _SEG + 2 * 32 - _K1 + 1
_K2 = 200
_L2 = 900 + 2 * 100 - _K2 + 1
_EPS = 1e-5
_BNS = float(1.0 / np.sqrt(1.0 + _EPS))


def _pool_mat(L, out):
    P = np.zeros((L, out), np.float32)
    for i in range(out):
        s = (i * L) // out
        e = -((-(i + 1) * L) // out)
        P[s:e, i] = 1.0 / (e - s)
    return P


_P1 = jnp.asarray(_pool_mat(_L1, 100))
_P2 = jnp.asarray(_pool_mat(_L2, 128))
_PEN = jnp.asarray(
    np.where(np.arange(_C * _C) % (_C + 1) == 0, -1e30, 0.0)
    .astype(np.float32).reshape(_C * _C, 1))


def _elu(z):
    return jnp.where(z > 0, z, jnp.exp(z) - 1.0)


def _conv1_body(x_ref, w_ref, g_ref, b_ref, p_ref, o_ref):
    x = x_ref[0]
    patch = jnp.concatenate([x[:, k:k + _L1] for k in range(_K1)], axis=0)
    z = jnp.dot(w_ref[...], patch, preferred_element_type=jnp.float32)
    z = z * g_ref[...] + b_ref[...]
    o_ref[0] = jnp.dot(_elu(z), p_ref[...], preferred_element_type=jnp.float32)


def _conv2_body(x_ref, w_ref, g_ref, b_ref, p_ref, o_ref):
    x = x_ref[0]
    z = jnp.zeros((_C, _L2), jnp.float32)
    step = 25
    for kb in range(0, _K2, step):
        patch = jnp.concatenate(
            [x[:, k:k + _L2] for k in range(kb, kb + step)], axis=0)
        z = z + jnp.dot(w_ref[:, kb * _C:(kb + step) * _C], patch,
                        preferred_element_type=jnp.float32)
    z = z * g_ref[...] + b_ref[...]
    o_ref[0] = jnp.dot(_elu(z), p_ref[...], preferred_element_type=jnp.float32)


def _ec_head_body(h_ref,
                  d1_ref, e1_ref, a1_ref, w21_ref, c1_ref, s1_ref, t1_ref,
                  d2_ref, e2_ref, a2_ref, w22_ref, c2_ref, s2_ref, t2_ref,
                  d3_ref, e3_ref, a3_ref, w23_ref, c3_ref, s3_ref, t3_ref,
                  l1a_ref, l1b_ref, l1c_ref, lb1_ref, l2_ref, lb2_ref,
                  pen_ref, o_ref):
    pen = pen_ref[...]

    def ec(h, d_r, e_r, b1_r, w2_r, b2_r, sc_r, be_r, F):
        A = jnp.dot(h, d_r[...], preferred_element_type=jnp.float32) + b1_r[...]
        Bm = jnp.dot(h, e_r[...], preferred_element_type=jnp.float32)
        Z = jnp.maximum(A[:, None, :] + Bm[None, :, :], 0.0)
        Z = Z.reshape(_C * _C, F)
        E = jnp.dot(Z, w2_r[...], preferred_element_type=jnp.float32)
        G = jnp.maximum(E + b2_r[...], 0.0) * sc_r[...] + be_r[...]
        G = G + pen
        return jnp.max(G.reshape(_C, _C, F), axis=1)

    h = h_ref[0]
    x1 = ec(h, d1_ref, e1_ref, a1_ref, w21_ref, c1_ref, s1_ref, t1_ref, 128)
    x2 = ec(x1, d2_ref, e2_ref, a2_ref, w22_ref, c2_ref, s2_ref, t2_ref, 256)
    x3 = ec(x2, d3_ref, e3_ref, a3_ref, w23_ref, c3_ref, s3_ref, t3_ref, 512)
    p1 = jnp.max(x1, axis=0, keepdims=True)
    p2 = jnp.max(x2, axis=0, keepdims=True)
    p3 = jnp.max(x3, axis=0, keepdims=True)
    q = (jnp.dot(p1, l1a_ref[...], preferred_element_type=jnp.float32)
         + jnp.dot(p2, l1b_ref[...], preferred_element_type=jnp.float32)
         + jnp.dot(p3, l1c_ref[...], preferred_element_type=jnp.float32)
         + lb1_ref[...])
    q = jnp.maximum(q, 0.0)
    o = jnp.dot(q, l2_ref[...], preferred_element_type=jnp.float32) + lb2_ref[...]
    o_ref[0] = jnp.maximum(o, 0.0)


def _full(shape):
    nd = len(shape)
    return pl.BlockSpec(shape, lambda i, _nd=nd: (0,) * _nd)


def kernel(x, conv1_w, bn1_g, bn1_b, conv2_w, bn2_g, bn2_b,
           c1_w1, c1_b1, c1_w2, c1_b2, c1_g, c1_be,
           c2_w1, c2_b1, c2_w2, c2_b2, c2_g, c2_be,
           c3_w1, c3_b1, c3_w2, c3_b2, c3_g, c3_be,
           lin1_w, lin1_b, lin2_w, lin2_b):
    xs = x.reshape(_B, _C, _NSEG, _SEG)
    xs = jnp.pad(xs, ((0, 0), (0, 0), (0, 0), (32, 32)))
    xs = xs.transpose(0, 2, 1, 3).reshape(_B * _NSEG, _C, _SEG + 64)
    w1r = conv1_w.transpose(0, 2, 1).reshape(_C, _K1 * _C)
    g1 = (bn1_g * _BNS).reshape(_C, 1)
    b1 = bn1_b.reshape(_C, 1)
    feats = pl.pallas_call(
        _conv1_body,
        grid=(_B * _NSEG,),
        in_specs=[
            pl.BlockSpec((1, _C, _SEG + 64), lambda i: (i, 0, 0)),
            _full(w1r.shape), _full(g1.shape), _full(b1.shape), _full(_P1.shape),
        ],
        out_specs=pl.BlockSpec((1, _C, 100), lambda i: (i, 0, 0)),
        out_shape=jax.ShapeDtypeStruct((_B * _NSEG, _C, 100), jnp.float32),
        compiler_params=pltpu.CompilerParams(
            dimension_semantics=("arbitrary",)),
    )(xs, w1r, g1, b1, _P1)
    h900 = feats.reshape(_B, _NSEG, _C, 100).transpose(0, 2, 1, 3)
    h900 = h900.reshape(_B, _C, 900)

    h900p = jnp.pad(h900, ((0, 0), (0, 0), (100, 100)))
    w2r = conv2_w.transpose(0, 2, 1).reshape(_C, _K2 * _C)
    g2 = (bn2_g * _BNS).reshape(_C, 1)
    b2 = bn2_b.reshape(_C, 1)
    h128 = pl.pallas_call(
        _conv2_body,
        grid=(_B,),
        in_specs=[
            pl.BlockSpec((1, _C, 1100), lambda i: (i, 0, 0)),
            _full(w2r.shape), _full(g2.shape), _full(b2.shape), _full(_P2.shape),
        ],
        out_specs=pl.BlockSpec((1, _C, 128), lambda i: (i, 0, 0)),
        out_shape=jax.ShapeDtypeStruct((_B, _C, 128), jnp.float32),
        compiler_params=pltpu.CompilerParams(
            dimension_semantics=("arbitrary",)),
    )(h900p, w2r, g2, b2, _P2)

    def ec_params(w1, b1v, w2, b2v, g, be, fin, F):
        d = (w1[:, :fin] - w1[:, fin:]).T
        e = w1[:, fin:].T
        return (d, e, b1v.reshape(1, F), w2.T,
                b2v.reshape(1, F), (g * _BNS).reshape(1, F), be.reshape(1, F))

    ec1 = ec_params(c1_w1, c1_b1, c1_w2, c1_b2, c1_g, c1_be, 128, 128)
    ec2 = ec_params(c2_w1, c2_b1, c2_w2, c2_b2, c2_g, c2_be, 128, 256)
    ec3 = ec_params(c3_w1, c3_b1, c3_w2, c3_b2, c3_g, c3_be, 256, 512)
    l1T = lin1_w.T
    l1a, l1b, l1c = l1T[:128], l1T[128:384], l1T[384:]
    lb1 = lin1_b.reshape(1, 256)
    l2T = lin2_w.T
    lb2 = lin2_b.reshape(1, 128)

    operands = (h128,) + ec1 + ec2 + ec3 + (l1a, l1b, l1c, lb1, l2T, lb2, _PEN)
    in_specs = [pl.BlockSpec((1, _C, 128), lambda i: (i, 0, 0))]
    in_specs += [_full(op.shape) for op in operands[1:]]
    out = pl.pallas_call(
        _ec_head_body,
        grid=(_B,),
        in_specs=in_specs,
        out_specs=pl.BlockSpec((1, 1, 128), lambda i: (i, 0, 0)),
        out_shape=jax.ShapeDtypeStruct((_B, 1, 128), jnp.float32),
        compiler_params=pltpu.CompilerParams(
            dimension_semantics=("arbitrary",)),
    )(*operands)
    return out.reshape(_B, 128)
